# baseline (speedup 1.0000x reference)
import functools

import jax
import jax.numpy as jnp
from jax import lax
from jax.experimental import pallas as pl
from jax.experimental.pallas import tpu as pltpu
from jax.experimental.pallas import tpu_sc as plsc

N = 10000
D = 128
DG = 16
E = 320000
NC = 2
NS = 16
NW = NC * NS
EPW = E // NW
CHUNK = 128
NCHUNK = 80
EPWP = NCHUNK * CHUNK
RPS = 632
NP = NS * RPS


def _sc_aggregate(h, src, dst, zeros_blk, zeros_deg, ones_deg, with_deg):
    mesh = plsc.VectorSubcoreMesh(core_axis_name="c", subcore_axis_name="s")

    out_type = [jax.ShapeDtypeStruct((NC, NP, D), jnp.bfloat16)]
    scratch = [
        pltpu.VMEM((NCHUNK, CHUNK), jnp.int32),
        pltpu.VMEM((NCHUNK, CHUNK), jnp.int32),
        pltpu.VMEM((2, CHUNK, D), jnp.bfloat16),
        pltpu.VMEM_SHARED((NP, D), jnp.bfloat16),
        pltpu.SemaphoreType.DMA((2,)),
        pltpu.SemaphoreType.DMA,
        pltpu.SemaphoreType.DMA,
    ]
    if with_deg:
        out_type.append(jax.ShapeDtypeStruct((NC, NP, DG), jnp.float32))
        scratch.append(pltpu.VMEM((CHUNK, DG), jnp.float32))
        scratch.append(pltpu.VMEM_SHARED((NP, DG), jnp.float32))

    @functools.partial(
        pl.kernel,
        out_type=out_type,
        mesh=mesh,
        scratch_types=scratch,
        compiler_params=pltpu.CompilerParams(use_tc_tiling_on_sc=False),
    )
    def agg(h_hbm, src_hbm, dst_hbm, z_hbm, zd_hbm, ones_hbm, *refs):
        if with_deg:
            (out_hbm, outd_hbm, src_v, dst_v, rows_v, acc_sh,
             gsem, sem1, sem2, ones_v, deg_sh) = refs
        else:
            out_hbm, src_v, dst_v, rows_v, acc_sh, gsem, sem1, sem2 = refs
        c = lax.axis_index("c")
        s = lax.axis_index("s")
        wid = c * NS + s

        pltpu.async_copy(z_hbm, acc_sh.at[pl.ds(s * RPS, RPS)], sem2)
        pltpu.async_copy(src_hbm.at[wid], src_v, gsem.at[0])
        pltpu.async_copy(dst_hbm.at[wid], dst_v, sem1)
        pltpu.make_async_copy(src_hbm.at[wid], src_v, gsem.at[0]).wait()
        pltpu.make_async_copy(dst_hbm.at[wid], dst_v, sem1).wait()
        if with_deg:
            pltpu.async_copy(zd_hbm, deg_sh.at[pl.ds(s * RPS, RPS)], sem1)
            pltpu.async_copy(ones_hbm, ones_v, gsem.at[1])
            pltpu.make_async_copy(ones_hbm, ones_v, gsem.at[1]).wait()
            pltpu.make_async_copy(zd_hbm, deg_sh.at[pl.ds(s * RPS, RPS)],
                                  sem1).wait()
        pltpu.make_async_copy(z_hbm, acc_sh.at[pl.ds(s * RPS, RPS)], sem2).wait()
        plsc.subcore_barrier()

        @pl.loop(0, NCHUNK + 1)
        def _(t):
            @pl.when(t < NCHUNK)
            def _():
                b = lax.rem(t, 2)
                pltpu.async_copy(h_hbm.at[src_v.at[t]], rows_v.at[b],
                                 gsem.at[b])

            @pl.when(t >= 1)
            def _():
                bp = lax.rem(t - 1, 2)
                pltpu.make_async_copy(h_hbm.at[src_v.at[t - 1]],
                                      rows_v.at[bp], gsem.at[bp]).wait()
                pltpu.sync_copy(rows_v.at[bp], acc_sh.at[dst_v.at[t - 1]],
                                add=True)
                if with_deg:
                    pltpu.sync_copy(ones_v, deg_sh.at[dst_v.at[t - 1]],
                                    add=True)

        plsc.subcore_barrier()
        pltpu.sync_copy(acc_sh.at[pl.ds(s * RPS, RPS)],
                        out_hbm.at[c, pl.ds(s * RPS, RPS)])
        if with_deg:
            pltpu.sync_copy(deg_sh.at[pl.ds(s * RPS, RPS)],
                            outd_hbm.at[c, pl.ds(s * RPS, RPS)])

    return agg(h, src, dst, zeros_blk, zeros_deg, ones_deg)


def _layer_body(h_ref, acc_ref, deg_ref, ws_ref, wn_ref, b_ref, *out_ref,
                relu):
    out_ref = out_ref[0] if len(out_ref) == 1 else out_ref
    h = h_ref[...]
    a = (acc_ref[0].astype(jnp.float32)
         + acc_ref[1].astype(jnp.float32))
    deg = jnp.maximum(deg_ref[0, :, 0:1] + deg_ref[1, :, 0:1], 1.0)
    hn = a / deg
    dn = (((1,), (1,)), ((), ()))
    out = lax.dot_general(h, ws_ref[...], dn,
                          preferred_element_type=jnp.float32)
    out = out + lax.dot_general(hn, wn_ref[...], dn,
                                preferred_element_type=jnp.float32)
    out = out + b_ref[...]
    if relu:
        out = jnp.maximum(out, 0.0)
    if isinstance(out_ref, (list, tuple)):
        out_ref[0][...] = out
        out_ref[1][...] = out.astype(jnp.bfloat16)
    else:
        out_ref[...] = out


def _tc_layer(h, acc, deg, W_self, W_neigh, b, *, relu, bf_out=False):
    BS = 1000
    out_specs = pl.BlockSpec((BS, D), lambda i: (i, 0))
    out_shape = jax.ShapeDtypeStruct((N, D), jnp.float32)
    if bf_out:
        out_specs = [out_specs, pl.BlockSpec((BS, D), lambda i: (i, 0))]
        out_shape = [out_shape, jax.ShapeDtypeStruct((N, D), jnp.bfloat16)]
    return pl.pallas_call(
        functools.partial(_layer_body, relu=relu),
        grid=(N // BS,),
        in_specs=[
            pl.BlockSpec((BS, D), lambda i: (i, 0)),
            pl.BlockSpec((NC, BS, D), lambda i: (0, i, 0)),
            pl.BlockSpec((NC, BS, DG), lambda i: (0, i, 0)),
            pl.BlockSpec((D, D), lambda i: (0, 0)),
            pl.BlockSpec((D, D), lambda i: (0, 0)),
            pl.BlockSpec((1, D), lambda i: (0, 0)),
        ],
        out_specs=out_specs,
        out_shape=out_shape,
    )(h, acc, deg, W_self, W_neigh, b)


def kernel(feat, edge_index, W_self0, W_neigh0, b0, W_self1, W_neigh1, b1):
    src = edge_index[0].astype(jnp.int32).reshape(NW, EPW)
    dst = edge_index[1].astype(jnp.int32).reshape(NW, EPW)
    pad = EPWP - EPW
    src = jnp.pad(src, ((0, 0), (0, pad))).reshape(NW, NCHUNK, CHUNK)
    pad_dst = jnp.broadcast_to(N + (jnp.arange(pad, dtype=jnp.int32) % (NP - N)),
                               (NW, pad))
    dst = jnp.concatenate([dst, pad_dst], axis=1).reshape(NW, NCHUNK, CHUNK)
    zeros_blk = jnp.zeros((RPS, D), jnp.bfloat16)
    zeros_deg = jnp.zeros((RPS, DG), jnp.float32)
    ones_deg = jnp.ones((CHUNK, DG), jnp.float32)
    b0r = b0.reshape(1, D)
    b1r = b1.reshape(1, D)

    feat_bf = feat.astype(jnp.bfloat16)
    acc0, deg = _sc_aggregate(feat_bf, src, dst, zeros_blk, zeros_deg,
                              ones_deg, with_deg=True)
    h1, h1_bf = _tc_layer(feat, acc0, deg, W_self0, W_neigh0, b0r, relu=True,
                          bf_out=True)
    acc1 = _sc_aggregate(h1_bf, src, dst, zeros_blk, zeros_deg, ones_deg,
                         with_deg=False)[0]
    out = _tc_layer(h1, acc1, deg, W_self1, W_neigh1, b1r, relu=False)
    return out


# speedup vs baseline: 1.5102x; 1.5102x over previous
import functools

import jax
import jax.numpy as jnp
from jax import lax
from jax.experimental import pallas as pl
from jax.experimental.pallas import tpu as pltpu
from jax.experimental.pallas import tpu_sc as plsc

N = 10000
D = 128
DG = 16
E = 320000
NC = 2
NS = 16
NW = NC * NS
EPW = E // NW
CHUNK = 120
NCHUNK = 84
EPWP = NCHUNK * CHUNK
RPS = 632
NP = NS * RPS


def _sc_aggregate(h, src, dst, zeros_blk, zeros_deg, ones_deg, with_deg):
    mesh = plsc.VectorSubcoreMesh(core_axis_name="c", subcore_axis_name="s")

    out_type = [jax.ShapeDtypeStruct((NC, NP, D), jnp.bfloat16)]
    scratch = [
        pltpu.VMEM((NCHUNK, CHUNK), jnp.int32),
        pltpu.VMEM((NCHUNK, CHUNK), jnp.int32),
        pltpu.VMEM((2, CHUNK, D), jnp.bfloat16),
        pltpu.VMEM_SHARED((NP, D), jnp.bfloat16),
        pltpu.SemaphoreType.DMA((2,)),
        pltpu.SemaphoreType.DMA,
        pltpu.SemaphoreType.DMA,
    ]
    if with_deg:
        out_type.append(jax.ShapeDtypeStruct((NC, NP, DG), jnp.float32))
        scratch.append(pltpu.VMEM((CHUNK, DG), jnp.float32))
        scratch.append(pltpu.VMEM_SHARED((NP, DG), jnp.float32))

    @functools.partial(
        pl.kernel,
        out_type=out_type,
        mesh=mesh,
        scratch_types=scratch,
        compiler_params=pltpu.CompilerParams(use_tc_tiling_on_sc=False),
    )
    def agg(h_hbm, src_hbm, dst_hbm, z_hbm, zd_hbm, ones_hbm, *refs):
        if with_deg:
            (out_hbm, outd_hbm, src_v, dst_v, rows_v, acc_sh,
             gsem, sem1, sem2, ones_v, deg_sh) = refs
        else:
            out_hbm, src_v, dst_v, rows_v, acc_sh, gsem, sem1, sem2 = refs
        c = lax.axis_index("c")
        s = lax.axis_index("s")
        wid = c * NS + s

        pltpu.async_copy(z_hbm, acc_sh.at[pl.ds(s * RPS, RPS)], sem2)
        pltpu.async_copy(src_hbm.at[wid], src_v, gsem.at[0])
        pltpu.async_copy(dst_hbm.at[wid], dst_v, sem1)
        pltpu.make_async_copy(src_hbm.at[wid], src_v, gsem.at[0]).wait()
        pltpu.make_async_copy(dst_hbm.at[wid], dst_v, sem1).wait()
        if with_deg:
            pltpu.async_copy(zd_hbm, deg_sh.at[pl.ds(s * RPS, RPS)], sem1)
            pltpu.async_copy(ones_hbm, ones_v, gsem.at[1])
            pltpu.make_async_copy(ones_hbm, ones_v, gsem.at[1]).wait()
            pltpu.make_async_copy(zd_hbm, deg_sh.at[pl.ds(s * RPS, RPS)],
                                  sem1).wait()
        pltpu.make_async_copy(z_hbm, acc_sh.at[pl.ds(s * RPS, RPS)], sem2).wait()
        plsc.subcore_barrier()

        @pl.loop(0, NCHUNK + 1)
        def _(t):
            @pl.when(t < NCHUNK)
            def _():
                b = lax.rem(t, 2)
                pltpu.async_copy(h_hbm.at[src_v.at[t]], rows_v.at[b],
                                 gsem.at[b])

            @pl.when(t >= 1)
            def _():
                bp = lax.rem(t - 1, 2)
                pltpu.make_async_copy(h_hbm.at[src_v.at[t - 1]],
                                      rows_v.at[bp], gsem.at[bp]).wait()
                pltpu.sync_copy(rows_v.at[bp], acc_sh.at[dst_v.at[t - 1]],
                                add=True)
                if with_deg:
                    pltpu.sync_copy(ones_v, deg_sh.at[dst_v.at[t - 1]],
                                    add=True)

        plsc.subcore_barrier()
        pltpu.sync_copy(acc_sh.at[pl.ds(s * RPS, RPS)],
                        out_hbm.at[c, pl.ds(s * RPS, RPS)])
        if with_deg:
            pltpu.sync_copy(deg_sh.at[pl.ds(s * RPS, RPS)],
                            outd_hbm.at[c, pl.ds(s * RPS, RPS)])

    return agg(h, src, dst, zeros_blk, zeros_deg, ones_deg)


def _layer_body(h_ref, acc_ref, deg_ref, ws_ref, wn_ref, b_ref, *out_ref,
                relu):
    out_ref = out_ref[0] if len(out_ref) == 1 else out_ref
    h = h_ref[...]
    a = (acc_ref[0].astype(jnp.float32)
         + acc_ref[1].astype(jnp.float32))
    deg = jnp.maximum(deg_ref[0, :, 0:1] + deg_ref[1, :, 0:1], 1.0)
    hn = a / deg
    dn = (((1,), (1,)), ((), ()))
    out = lax.dot_general(h, ws_ref[...], dn,
                          preferred_element_type=jnp.float32)
    out = out + lax.dot_general(hn, wn_ref[...], dn,
                                preferred_element_type=jnp.float32)
    out = out + b_ref[...]
    if relu:
        out = jnp.maximum(out, 0.0)
    if isinstance(out_ref, (list, tuple)):
        out_ref[0][...] = out
        out_ref[1][...] = out.astype(jnp.bfloat16)
    else:
        out_ref[...] = out


def _tc_layer(h, acc, deg, W_self, W_neigh, b, *, relu, bf_out=False):
    BS = 1000
    out_specs = pl.BlockSpec((BS, D), lambda i: (i, 0))
    out_shape = jax.ShapeDtypeStruct((N, D), jnp.float32)
    if bf_out:
        out_specs = [out_specs, pl.BlockSpec((BS, D), lambda i: (i, 0))]
        out_shape = [out_shape, jax.ShapeDtypeStruct((N, D), jnp.bfloat16)]
    return pl.pallas_call(
        functools.partial(_layer_body, relu=relu),
        grid=(N // BS,),
        in_specs=[
            pl.BlockSpec((BS, D), lambda i: (i, 0)),
            pl.BlockSpec((NC, BS, D), lambda i: (0, i, 0)),
            pl.BlockSpec((NC, BS, DG), lambda i: (0, i, 0)),
            pl.BlockSpec((D, D), lambda i: (0, 0)),
            pl.BlockSpec((D, D), lambda i: (0, 0)),
            pl.BlockSpec((1, D), lambda i: (0, 0)),
        ],
        out_specs=out_specs,
        out_shape=out_shape,
    )(h, acc, deg, W_self, W_neigh, b)


def kernel(feat, edge_index, W_self0, W_neigh0, b0, W_self1, W_neigh1, b1):
    src = edge_index[0].astype(jnp.int32).reshape(NW, EPW)
    dst = edge_index[1].astype(jnp.int32).reshape(NW, EPW)
    pad = EPWP - EPW
    src = jnp.pad(src, ((0, 0), (0, pad))).reshape(NW, NCHUNK, CHUNK)
    pad_dst = jnp.broadcast_to(N + (jnp.arange(pad, dtype=jnp.int32) % (NP - N)),
                               (NW, pad))
    dst = jnp.concatenate([dst, pad_dst], axis=1).reshape(NW, NCHUNK, CHUNK)
    zeros_blk = jnp.zeros((RPS, D), jnp.bfloat16)
    zeros_deg = jnp.zeros((RPS, DG), jnp.float32)
    ones_deg = jnp.ones((CHUNK, DG), jnp.float32)
    b0r = b0.reshape(1, D)
    b1r = b1.reshape(1, D)

    feat_bf = feat.astype(jnp.bfloat16)
    acc0, deg = _sc_aggregate(feat_bf, src, dst, zeros_blk, zeros_deg,
                              ones_deg, with_deg=True)
    h1, h1_bf = _tc_layer(feat, acc0, deg, W_self0, W_neigh0, b0r, relu=True,
                          bf_out=True)
    acc1 = _sc_aggregate(h1_bf, src, dst, zeros_blk, zeros_deg, ones_deg,
                         with_deg=False)[0]
    out = _tc_layer(h1, acc1, deg, W_self1, W_neigh1, b1r, relu=False)
    return out
